# baseline (device time: 18025 ns/iter reference)
import jax
import jax.numpy as jnp
from jax import lax
from jax.experimental import pallas as pl
from jax.experimental.pallas import tpu as pltpu


def kernel(x):
    m, n = x.shape
    half = m // 2

    def body(x_ref, out_ref, send_b, recv_b, s1, r1, s2, r2):
        my_x = lax.axis_index("x")
        my_y = lax.axis_index("y")
        my_z = lax.axis_index("z")
        xpeer = (1 - my_x, my_y, my_z)

        barrier = pltpu.get_barrier_semaphore()
        pl.semaphore_signal(
            barrier, inc=1, device_id=xpeer, device_id_type=pl.DeviceIdType.MESH
        )
        pl.semaphore_wait(barrier, 1)

        send_b[...] = x_ref[...].astype(jnp.bfloat16)
        rd1 = pltpu.make_async_remote_copy(
            src_ref=send_b.at[pl.ds(0, half), :],
            dst_ref=recv_b.at[pl.ds(0, half), :],
            send_sem=s1,
            recv_sem=r1,
            device_id=xpeer,
            device_id_type=pl.DeviceIdType.MESH,
        )
        rd2 = pltpu.make_async_remote_copy(
            src_ref=send_b.at[pl.ds(half, half), :],
            dst_ref=recv_b.at[pl.ds(half, half), :],
            send_sem=s2,
            recv_sem=r2,
            device_id=xpeer,
            device_id_type=pl.DeviceIdType.MESH,
        )
        rd1.start()
        rd2.start()
        rd1.wait()
        rd2.wait()
        out_ref[...] = send_b[...] + recv_b[...]

    return pl.pallas_call(
        body,
        out_shape=jax.ShapeDtypeStruct((m, n), jnp.bfloat16),
        in_specs=[pl.BlockSpec(memory_space=pltpu.VMEM)],
        out_specs=pl.BlockSpec(memory_space=pltpu.VMEM),
        scratch_shapes=[
            pltpu.VMEM((m, n), jnp.bfloat16),
            pltpu.VMEM((m, n), jnp.bfloat16),
            pltpu.SemaphoreType.DMA,
            pltpu.SemaphoreType.DMA,
            pltpu.SemaphoreType.DMA,
            pltpu.SemaphoreType.DMA,
        ],
        compiler_params=pltpu.CompilerParams(collective_id=0),
    )(x)


# device time: 16981 ns/iter; 1.0615x vs baseline; 1.0615x over previous
import jax
import jax.numpy as jnp
from jax import lax
from jax.experimental import pallas as pl
from jax.experimental.pallas import tpu as pltpu

C = 8


def kernel(x):
    m, n = x.shape
    half = m // 2
    ch = half // C

    def body(
        x_hbm,
        out_ref,
        xh,
        send_x,
        recv_x,
        cp_sems,
        sx_sems,
        rx_sems,
        sz_sems,
        rz_sems,
    ):
        my_x = lax.axis_index("x")
        my_y = lax.axis_index("y")
        my_z = lax.axis_index("z")
        xpeer = (1 - my_x, my_y, my_z)
        zpeer = (my_x, my_y, 1 - my_z)

        base = my_z * half

        cps = []
        for c in range(C):
            r0 = c * ch
            cp = pltpu.make_async_copy(
                x_hbm.at[pl.ds(base + r0, ch), :],
                xh.at[pl.ds(r0, ch), :],
                cp_sems.at[c],
            )
            cp.start()
            cps.append(cp)

        barrier = pltpu.get_barrier_semaphore()
        for nbr in (xpeer, zpeer):
            pl.semaphore_signal(
                barrier, inc=1, device_id=nbr, device_id_type=pl.DeviceIdType.MESH
            )
        pl.semaphore_wait(barrier, 2)

        rd_a = []
        for c in range(C):
            r0 = c * ch
            cps[c].wait()
            send_x[pl.ds(r0, ch), :] = xh[pl.ds(r0, ch), :].astype(jnp.bfloat16)
            rd = pltpu.make_async_remote_copy(
                src_ref=send_x.at[pl.ds(r0, ch), :],
                dst_ref=recv_x.at[pl.ds(r0, ch), :],
                send_sem=sx_sems.at[c],
                recv_sem=rx_sems.at[c],
                device_id=xpeer,
                device_id_type=pl.DeviceIdType.MESH,
            )
            rd.start()
            rd_a.append(rd)

        rd_b = []
        for c in range(C):
            r0 = c * ch
            rd_a[c].wait_recv()
            out_ref[pl.ds(base + r0, ch), :] = (
                send_x[pl.ds(r0, ch), :] + recv_x[pl.ds(r0, ch), :]
            )
            rd = pltpu.make_async_remote_copy(
                src_ref=out_ref.at[pl.ds(base + r0, ch), :],
                dst_ref=out_ref.at[pl.ds(base + r0, ch), :],
                send_sem=sz_sems.at[c],
                recv_sem=rz_sems.at[c],
                device_id=zpeer,
                device_id_type=pl.DeviceIdType.MESH,
            )
            rd.start()
            rd_b.append(rd)

        for c in range(C):
            rd_b[c].wait_recv()
        for c in range(C):
            rd_a[c].wait_send()
            rd_b[c].wait_send()

    return pl.pallas_call(
        body,
        out_shape=jax.ShapeDtypeStruct((m, n), jnp.bfloat16),
        in_specs=[pl.BlockSpec(memory_space=pl.ANY)],
        out_specs=pl.BlockSpec(memory_space=pltpu.VMEM),
        scratch_shapes=[
            pltpu.VMEM((half, n), jnp.float32),
            pltpu.VMEM((half, n), jnp.bfloat16),
            pltpu.VMEM((half, n), jnp.bfloat16),
            pltpu.SemaphoreType.DMA((C,)),
            pltpu.SemaphoreType.DMA((C,)),
            pltpu.SemaphoreType.DMA((C,)),
            pltpu.SemaphoreType.DMA((C,)),
            pltpu.SemaphoreType.DMA((C,)),
        ],
        compiler_params=pltpu.CompilerParams(collective_id=0),
    )(x)


# device time: 15493 ns/iter; 1.1634x vs baseline; 1.0960x over previous
import jax
import jax.numpy as jnp
from jax import lax
from jax.experimental import pallas as pl
from jax.experimental.pallas import tpu as pltpu

C = 8


def kernel(x):
    m, n = x.shape
    half = m // 2
    ch = half // C

    def body(
        x_ref,
        out_ref,
        send_x,
        recv_x,
        zsem,
        sx_sems,
        rx_sems,
        sz_sems,
        rz_sems,
    ):
        my_x = lax.axis_index("x")
        my_y = lax.axis_index("y")
        my_z = lax.axis_index("z")
        xpeer = (1 - my_x, my_y, my_z)
        zpeer = (my_x, my_y, 1 - my_z)

        barrier = pltpu.get_barrier_semaphore()
        pl.semaphore_signal(
            barrier, inc=1, device_id=xpeer, device_id_type=pl.DeviceIdType.MESH
        )
        pl.semaphore_signal(
            zsem, inc=1, device_id=zpeer, device_id_type=pl.DeviceIdType.MESH
        )
        pl.semaphore_wait(barrier, 1)

        base = my_z * half

        rd_a = []
        for c in range(C):
            r0 = c * ch
            send_x[pl.ds(r0, ch), :] = x_ref[pl.ds(base + r0, ch), :].astype(
                jnp.bfloat16
            )
            rd = pltpu.make_async_remote_copy(
                src_ref=send_x.at[pl.ds(r0, ch), :],
                dst_ref=recv_x.at[pl.ds(r0, ch), :],
                send_sem=sx_sems.at[c],
                recv_sem=rx_sems.at[c],
                device_id=xpeer,
                device_id_type=pl.DeviceIdType.MESH,
            )
            rd.start()
            rd_a.append(rd)

        rd_b = []
        for c in range(C):
            r0 = c * ch
            rd_a[c].wait_recv()
            out_ref[pl.ds(base + r0, ch), :] = (
                send_x[pl.ds(r0, ch), :] + recv_x[pl.ds(r0, ch), :]
            )
            if c == 0:
                pl.semaphore_wait(zsem, 1)
            rd = pltpu.make_async_remote_copy(
                src_ref=out_ref.at[pl.ds(base + r0, ch), :],
                dst_ref=out_ref.at[pl.ds(base + r0, ch), :],
                send_sem=sz_sems.at[c],
                recv_sem=rz_sems.at[c],
                device_id=zpeer,
                device_id_type=pl.DeviceIdType.MESH,
            )
            rd.start()
            rd_b.append(rd)

        for c in range(C):
            rd_b[c].wait_recv()
        for c in range(C):
            rd_a[c].wait_send()
            rd_b[c].wait_send()

    return pl.pallas_call(
        body,
        out_shape=jax.ShapeDtypeStruct((m, n), jnp.bfloat16),
        in_specs=[pl.BlockSpec(memory_space=pltpu.VMEM)],
        out_specs=pl.BlockSpec(memory_space=pltpu.VMEM),
        scratch_shapes=[
            pltpu.VMEM((half, n), jnp.bfloat16),
            pltpu.VMEM((half, n), jnp.bfloat16),
            pltpu.SemaphoreType.REGULAR,
            pltpu.SemaphoreType.DMA((C,)),
            pltpu.SemaphoreType.DMA((C,)),
            pltpu.SemaphoreType.DMA((C,)),
            pltpu.SemaphoreType.DMA((C,)),
        ],
        compiler_params=pltpu.CompilerParams(collective_id=0),
    )(x)


# device time: 15413 ns/iter; 1.1695x vs baseline; 1.0052x over previous
import jax
import jax.numpy as jnp
from jax import lax
from jax.experimental import pallas as pl
from jax.experimental.pallas import tpu as pltpu

C = 16


def kernel(x):
    m, n = x.shape
    half = m // 2
    ch = half // C

    def body(
        x_ref,
        out_ref,
        send_x,
        recv_x,
        zsem,
        sx_sems,
        rx_sems,
        sz_sems,
        rz_sems,
    ):
        my_x = lax.axis_index("x")
        my_y = lax.axis_index("y")
        my_z = lax.axis_index("z")
        xpeer = (1 - my_x, my_y, my_z)
        zpeer = (my_x, my_y, 1 - my_z)

        barrier = pltpu.get_barrier_semaphore()
        pl.semaphore_signal(
            barrier, inc=1, device_id=xpeer, device_id_type=pl.DeviceIdType.MESH
        )
        pl.semaphore_signal(
            zsem, inc=1, device_id=zpeer, device_id_type=pl.DeviceIdType.MESH
        )
        pl.semaphore_wait(barrier, 1)

        base = my_z * half

        rd_a = []
        for c in range(C):
            r0 = c * ch
            send_x[pl.ds(r0, ch), :] = x_ref[pl.ds(base + r0, ch), :].astype(
                jnp.bfloat16
            )
            rd = pltpu.make_async_remote_copy(
                src_ref=send_x.at[pl.ds(r0, ch), :],
                dst_ref=recv_x.at[pl.ds(r0, ch), :],
                send_sem=sx_sems.at[c],
                recv_sem=rx_sems.at[c],
                device_id=xpeer,
                device_id_type=pl.DeviceIdType.MESH,
            )
            rd.start()
            rd_a.append(rd)

        rd_b = []
        for c in range(C):
            r0 = c * ch
            rd_a[c].wait_recv()
            out_ref[pl.ds(base + r0, ch), :] = (
                send_x[pl.ds(r0, ch), :] + recv_x[pl.ds(r0, ch), :]
            )
            if c == 0:
                pl.semaphore_wait(zsem, 1)
            rd = pltpu.make_async_remote_copy(
                src_ref=out_ref.at[pl.ds(base + r0, ch), :],
                dst_ref=out_ref.at[pl.ds(base + r0, ch), :],
                send_sem=sz_sems.at[c],
                recv_sem=rz_sems.at[c],
                device_id=zpeer,
                device_id_type=pl.DeviceIdType.MESH,
            )
            rd.start()
            rd_b.append(rd)

        for c in range(C):
            rd_b[c].wait_recv()
        for c in range(C):
            rd_a[c].wait_send()
            rd_b[c].wait_send()

    return pl.pallas_call(
        body,
        out_shape=jax.ShapeDtypeStruct((m, n), jnp.bfloat16),
        in_specs=[pl.BlockSpec(memory_space=pltpu.VMEM)],
        out_specs=pl.BlockSpec(memory_space=pltpu.VMEM),
        scratch_shapes=[
            pltpu.VMEM((half, n), jnp.bfloat16),
            pltpu.VMEM((half, n), jnp.bfloat16),
            pltpu.SemaphoreType.REGULAR,
            pltpu.SemaphoreType.DMA((C,)),
            pltpu.SemaphoreType.DMA((C,)),
            pltpu.SemaphoreType.DMA((C,)),
            pltpu.SemaphoreType.DMA((C,)),
        ],
        compiler_params=pltpu.CompilerParams(collective_id=0),
    )(x)
